# baseline (speedup 1.0000x reference)
import jax
import jax.numpy as jnp
from jax import lax
from jax.experimental import pallas as pl
from jax.experimental.pallas import tpu as pltpu
from jax.experimental.pallas import tpu_sc as plsc

N = 10000
D = 128
E = 320000

NC = 2
NS = 16
L = 16

DH = D // NC
NP = 10240
CHUNK = 128
EPT = E // NS
G = 4
NGRP = 40
NCHUNK = NGRP * G
EPTP = NCHUNK * CHUNK
RPT = NP // NS
RCH = 128
NRCH = RPT // RCH

NBUF = 4
AHEAD = 2
NSETS = 4


def _body(table_hbm, dst2_hbm, src_hbm, out_hbm,
          dst_idx_v, src_idx_v, rows_v, div_v, ddeg_v, ones_v,
          acc_s, deg_s, gsem, ssem, dsem, isem):
    c = lax.axis_index("c")
    s = lax.axis_index("s")
    row0 = s * RPT

    def iset(chunk):
        return (chunk // G) % NSETS

    def fire_idx(grp):
        st = grp % NSETS
        pltpu.async_copy(dst2_hbm.at[c, s, grp], dst_idx_v.at[st], isem.at[st])
        pltpu.async_copy(src_hbm.at[s, grp], src_idx_v.at[st], isem.at[st])

    def wait_idx(grp):
        st = grp % NSETS
        pltpu.make_async_copy(dst2_hbm.at[c, s, 0], dst_idx_v.at[st],
                              isem.at[st]).wait()
        pltpu.make_async_copy(src_hbm.at[s, 0], src_idx_v.at[st],
                              isem.at[st]).wait()

    def fire_gather(ch, row, b):
        pltpu.async_copy(table_hbm.at[dst_idx_v.at[iset(ch), row]],
                         rows_v.at[b], gsem.at[b])

    def wait_gather(b):
        pltpu.make_async_copy(table_hbm.at[dst_idx_v.at[0, 0]], rows_v.at[b],
                              gsem.at[b]).wait()

    def fire_scatter(ch, row, b):
        idx = src_idx_v.at[iset(ch), row]
        pltpu.async_copy(rows_v.at[b], acc_s.at[idx], ssem.at[b], add=True)
        pltpu.async_copy(ones_v, deg_s.at[idx], dsem.at[b], add=True)

    def wait_scatter(b):
        pltpu.make_async_copy(rows_v.at[b], acc_s.at[src_idx_v.at[0, 0]],
                              ssem.at[b]).wait()
        pltpu.make_async_copy(ones_v, deg_s.at[src_idx_v.at[0, 0]],
                              dsem.at[b]).wait()

    def _zero_row(r, _):
        for k in range(DH // L):
            div_v[r, pl.ds(k * L, L)] = jnp.zeros((L,), jnp.float32)
        ddeg_v[r, :] = jnp.zeros((L,), jnp.float32)
        ones_v[r, :] = jnp.ones((L,), jnp.float32)
        return _
    lax.fori_loop(0, RCH, _zero_row, None)
    for k in range(NRCH):
        pltpu.sync_copy(div_v, acc_s.at[pl.ds(row0 + k * RCH, RCH)])
        pltpu.sync_copy(ddeg_v, deg_s.at[pl.ds(row0 + k * RCH, RCH)])

    for grp in range(3):
        fire_idx(grp)
    wait_idx(0)

    plsc.subcore_barrier()

    for ch in range(AHEAD):
        fire_gather(ch, ch % G, ch % NBUF)
    for ch in range(NBUF):
        if ch == AHEAD:
            wait_idx(1)
        wait_gather(ch)
        fire_scatter(ch, ch % G, ch)
        if ch + AHEAD >= NBUF:
            wait_scatter((ch + AHEAD) % NBUF)
        fire_gather(ch + AHEAD, (ch + AHEAD) % G, (ch + AHEAD) % NBUF)

    def _group(i, _):
        g = i + 1
        j = g * G

        @pl.when(g + 2 < NGRP)
        def _():
            fire_idx(g + 2)

        for b in range(NBUF):
            ch = j + b
            if b == AHEAD:
                @pl.when(g + 1 < NGRP)
                def _():
                    wait_idx(g + 1)
            wait_gather(b)
            fire_scatter(ch, b, b)
            nxt = ch + AHEAD

            @pl.when(nxt < NCHUNK)
            def _():
                b2 = (b + AHEAD) % NBUF
                wait_scatter(b2)
                fire_gather(nxt, (b + AHEAD) % G, b2)
        return _
    lax.fori_loop(0, NGRP - 1, _group, None)

    for b in range(NBUF):
        wait_scatter(b)

    plsc.subcore_barrier()

    for k in range(NRCH):
        base = row0 + k * RCH
        pltpu.sync_copy(acc_s.at[pl.ds(base, RCH)], div_v)
        pltpu.sync_copy(deg_s.at[pl.ds(base, RCH)], ddeg_v)

        def _div_row(r, _):
            recip = 1.0 / ddeg_v[r, :]
            for q in range(DH // L):
                div_v[r, pl.ds(q * L, L)] = div_v[r, pl.ds(q * L, L)] * recip
            return _
        lax.fori_loop(0, RCH, _div_row, None)
        pltpu.sync_copy(div_v, out_hbm.at[pl.ds(base, RCH), pl.ds(c * DH, DH)])


def kernel(features, edge_index):
    src = edge_index[0]
    dst = edge_index[1]
    table = features.reshape(N, NC, DH).transpose(1, 0, 2).reshape(NC * N, DH)
    pad = ((0, 0), (0, 0), (0, EPTP - EPT))
    dst2 = jnp.pad(jnp.stack([dst, dst + N]).reshape(NC, NS, EPT), pad)
    dst2 = dst2.reshape(NC, NS, NGRP, G, CHUNK)
    src_r = jnp.pad(src.reshape(1, NS, EPT), pad, constant_values=NP - 1)
    src_r = src_r.reshape(NS, NGRP, G, CHUNK)

    mesh = plsc.VectorSubcoreMesh(core_axis_name="c", subcore_axis_name="s")
    k = pl.kernel(
        _body,
        out_type=jax.ShapeDtypeStruct((NP, D), jnp.float32),
        mesh=mesh,
        scratch_types=[
            pltpu.VMEM((NSETS, G, CHUNK), jnp.int32),
            pltpu.VMEM((NSETS, G, CHUNK), jnp.int32),
            pltpu.VMEM((NBUF, CHUNK, DH), jnp.float32),
            pltpu.VMEM((RCH, DH), jnp.float32),
            pltpu.VMEM((RCH, L), jnp.float32),
            pltpu.VMEM((CHUNK, L), jnp.float32),
            pltpu.VMEM_SHARED((NP, DH), jnp.float32),
            pltpu.VMEM_SHARED((NP, L), jnp.float32),
            pltpu.SemaphoreType.DMA((NBUF,)),
            pltpu.SemaphoreType.DMA((NBUF,)),
            pltpu.SemaphoreType.DMA((NBUF,)),
            pltpu.SemaphoreType.DMA((NSETS,)),
        ],
        compiler_params=pltpu.CompilerParams(use_tc_tiling_on_sc=False),
    )
    return k(table, dst2, src_r)[:N]


# speedup vs baseline: 2.1987x; 2.1987x over previous
import jax
import jax.numpy as jnp
from jax import lax
from jax.experimental import pallas as pl
from jax.experimental.pallas import tpu as pltpu
from jax.experimental.pallas import tpu_sc as plsc

N = 10000
D = 128
E = 320000

NC = 2
NS = 16
L = 16

DH = D // NC
W = DH + L
EPT = E // NS
CHUNK = 125
NCHUNK = EPT // CHUNK
RPT = N // NS
RCH = 125
NRCH = RPT // RCH

NBUF = 4
AHEAD = 2
G = 4
NGRP = NCHUNK // G
NSETS = 4


def _body(table_hbm, dst2_hbm, src_hbm, out_hbm,
          dst_idx_v, src_idx_v, rows_v, div_v, out_v, acc_s,
          gsem, ssem, isem):
    c = lax.axis_index("c")
    s = lax.axis_index("s")
    row0 = s * RPT

    def iset(chunk):
        return (chunk // G) % NSETS

    def fire_idx(grp):
        st = grp % NSETS
        pltpu.async_copy(dst2_hbm.at[c, s, grp], dst_idx_v.at[st], isem.at[st])
        pltpu.async_copy(src_hbm.at[s, grp], src_idx_v.at[st], isem.at[st])

    def wait_idx(grp):
        st = grp % NSETS
        pltpu.make_async_copy(dst2_hbm.at[c, s, 0], dst_idx_v.at[st],
                              isem.at[st]).wait()
        pltpu.make_async_copy(src_hbm.at[s, 0], src_idx_v.at[st],
                              isem.at[st]).wait()

    def fire_gather(ch, row, b):
        pltpu.async_copy(table_hbm.at[dst_idx_v.at[iset(ch), row]],
                         rows_v.at[b], gsem.at[b])

    def wait_gather(b):
        pltpu.make_async_copy(table_hbm.at[dst_idx_v.at[0, 0]], rows_v.at[b],
                              gsem.at[b]).wait()

    def fire_scatter(ch, row, b):
        pltpu.async_copy(rows_v.at[b], acc_s.at[src_idx_v.at[iset(ch), row]],
                         ssem.at[b], add=True)

    def wait_scatter(b):
        pltpu.make_async_copy(rows_v.at[b], acc_s.at[src_idx_v.at[0, 0]],
                              ssem.at[b]).wait()

    def _zero_row(r, _):
        for k in range(W // L):
            div_v[r, pl.ds(k * L, L)] = jnp.zeros((L,), jnp.float32)
        return _
    lax.fori_loop(0, RCH, _zero_row, None)
    for k in range(NRCH):
        pltpu.sync_copy(div_v, acc_s.at[pl.ds(row0 + k * RCH, RCH)])

    for grp in range(min(3, NGRP)):
        fire_idx(grp)
    wait_idx(0)

    plsc.subcore_barrier()

    for ch in range(AHEAD):
        fire_gather(ch, ch % G, ch % NBUF)
    for ch in range(NBUF):
        if ch == AHEAD:
            wait_idx(1)
        wait_gather(ch)
        fire_scatter(ch, ch % G, ch)
        if ch + AHEAD >= NBUF:
            wait_scatter((ch + AHEAD) % NBUF)
        fire_gather(ch + AHEAD, (ch + AHEAD) % G, (ch + AHEAD) % NBUF)

    def _group(i, _):
        g = i + 1
        j = g * G

        @pl.when(g + 2 < NGRP)
        def _():
            fire_idx(g + 2)

        for b in range(NBUF):
            ch = j + b
            if b == AHEAD:
                @pl.when(g + 1 < NGRP)
                def _():
                    wait_idx(g + 1)
            wait_gather(b)
            fire_scatter(ch, b, b)
            nxt = ch + AHEAD

            @pl.when(nxt < NCHUNK)
            def _():
                b2 = (b + AHEAD) % NBUF
                wait_scatter(b2)
                fire_gather(nxt, (b + AHEAD) % G, b2)
        return _
    lax.fori_loop(0, NGRP - 1, _group, None)

    for b in range(NBUF):
        wait_scatter(b)

    plsc.subcore_barrier()

    for k in range(NRCH):
        base = row0 + k * RCH
        pltpu.sync_copy(acc_s.at[pl.ds(base, RCH)], div_v)

        def _div_row(r, _):
            deg = div_v[r, pl.ds(DH, L)]
            recip = 1.0 / deg
            for q in range(DH // L):
                out_v[r, pl.ds(q * L, L)] = div_v[r, pl.ds(q * L, L)] * recip
            return _
        lax.fori_loop(0, RCH, _div_row, None)
        pltpu.sync_copy(out_v, out_hbm.at[pl.ds(base, RCH), pl.ds(c * DH, DH)])


def kernel(features, edge_index):
    src = edge_index[0]
    dst = edge_index[1]
    ones = jnp.ones((N, L), jnp.float32)
    table = jnp.concatenate([
        jnp.concatenate([features[:, :DH], ones], axis=1),
        jnp.concatenate([features[:, DH:], ones], axis=1),
    ], axis=0)
    dst2 = jnp.stack([dst, dst + N]).reshape(NC, NS, NGRP, G, CHUNK)
    src_r = src.reshape(NS, NGRP, G, CHUNK)

    mesh = plsc.VectorSubcoreMesh(core_axis_name="c", subcore_axis_name="s")
    k = pl.kernel(
        _body,
        out_type=jax.ShapeDtypeStruct((N, D), jnp.float32),
        mesh=mesh,
        scratch_types=[
            pltpu.VMEM((NSETS, G, CHUNK), jnp.int32),
            pltpu.VMEM((NSETS, G, CHUNK), jnp.int32),
            pltpu.VMEM((NBUF, CHUNK, W), jnp.float32),
            pltpu.VMEM((RCH, W), jnp.float32),
            pltpu.VMEM((RCH, DH), jnp.float32),
            pltpu.VMEM_SHARED((N, W), jnp.float32),
            pltpu.SemaphoreType.DMA((NBUF,)),
            pltpu.SemaphoreType.DMA((NBUF,)),
            pltpu.SemaphoreType.DMA((NSETS,)),
        ],
        compiler_params=pltpu.CompilerParams(use_tc_tiling_on_sc=False),
    )
    return k(table, dst2, src_r)


# speedup vs baseline: 2.3189x; 1.0547x over previous
import jax
import jax.numpy as jnp
from jax import lax
from jax.experimental import pallas as pl
from jax.experimental.pallas import tpu as pltpu
from jax.experimental.pallas import tpu_sc as plsc

N = 10000
D = 128
E = 320000

NC = 2
NS = 16
L = 16

DH = D // NC
W = DH + L
EPT = E // NS
CHUNK = 125
NCHUNK = EPT // CHUNK
RPT = N // NS
RCH = 125
NRCH = RPT // RCH

NBUF = 4
AHEAD = 2
G = 4
NGRP = NCHUNK // G
NSETS = 4


def _body(feat_hbm, dst_hbm, src_hbm, out_hbm, table_hbm,
          dst_idx_v, src_idx_v, rows_v, div_v, out_v, feat_v, acc_s,
          gsem, ssem, isem):
    c = lax.axis_index("c")
    s = lax.axis_index("s")
    row0 = s * RPT

    def iset(chunk):
        return (chunk // G) % NSETS

    def fire_idx(grp):
        st = grp % NSETS
        pltpu.async_copy(dst_hbm.at[s, grp], dst_idx_v.at[st], isem.at[st])
        pltpu.async_copy(src_hbm.at[s, grp], src_idx_v.at[st], isem.at[st])

    def wait_idx(grp):
        st = grp % NSETS
        pltpu.make_async_copy(dst_hbm.at[s, 0], dst_idx_v.at[st],
                              isem.at[st]).wait()
        pltpu.make_async_copy(src_hbm.at[s, 0], src_idx_v.at[st],
                              isem.at[st]).wait()

    tab_core = table_hbm.at[pl.ds(c * N, N)]

    def fire_gather(ch, row, b):
        pltpu.async_copy(tab_core.at[dst_idx_v.at[iset(ch), row]],
                         rows_v.at[b], gsem.at[b])

    def wait_gather(b):
        pltpu.make_async_copy(tab_core.at[dst_idx_v.at[0, 0]], rows_v.at[b],
                              gsem.at[b]).wait()

    def fire_scatter(ch, row, b):
        pltpu.async_copy(rows_v.at[b], acc_s.at[src_idx_v.at[iset(ch), row]],
                         ssem.at[b], add=True)

    def wait_scatter(b):
        pltpu.make_async_copy(rows_v.at[b], acc_s.at[src_idx_v.at[0, 0]],
                              ssem.at[b]).wait()

    def _ones_row(r, _):
        div_v[r, pl.ds(DH, L)] = jnp.ones((L,), jnp.float32)
        return _
    lax.fori_loop(0, RCH, _ones_row, None)

    def _build(cc):
        for k in range(NRCH):
            pltpu.sync_copy(feat_hbm.at[pl.ds(row0 + k * RCH, RCH)], feat_v)

            def _cp_row(r, _):
                for q in range(DH // L):
                    div_v[r, pl.ds(q * L, L)] = \
                        feat_v[r, pl.ds(cc * DH + q * L, L)]
                return _
            lax.fori_loop(0, RCH, _cp_row, None)
            pltpu.sync_copy(
                div_v, table_hbm.at[pl.ds(cc * N + row0 + k * RCH, RCH)])

    @pl.when(c == 0)
    def _():
        _build(0)

    @pl.when(c == 1)
    def _():
        _build(1)

    def _zero_row(r, _):
        for k in range(W // L):
            div_v[r, pl.ds(k * L, L)] = jnp.zeros((L,), jnp.float32)
        return _
    lax.fori_loop(0, RCH, _zero_row, None)
    for k in range(NRCH):
        pltpu.sync_copy(div_v, acc_s.at[pl.ds(row0 + k * RCH, RCH)])

    for grp in range(min(3, NGRP)):
        fire_idx(grp)
    wait_idx(0)

    plsc.subcore_barrier()

    for ch in range(AHEAD):
        fire_gather(ch, ch % G, ch % NBUF)
    for ch in range(NBUF):
        if ch == AHEAD:
            wait_idx(1)
        wait_gather(ch)
        fire_scatter(ch, ch % G, ch)
        if ch + AHEAD >= NBUF:
            wait_scatter((ch + AHEAD) % NBUF)
        fire_gather(ch + AHEAD, (ch + AHEAD) % G, (ch + AHEAD) % NBUF)

    def _group(i, _):
        g = i + 1
        j = g * G

        @pl.when(g + 2 < NGRP)
        def _():
            fire_idx(g + 2)

        for b in range(NBUF):
            ch = j + b
            if b == AHEAD:
                @pl.when(g + 1 < NGRP)
                def _():
                    wait_idx(g + 1)
            wait_gather(b)
            fire_scatter(ch, b, b)
            nxt = ch + AHEAD

            @pl.when(nxt < NCHUNK)
            def _():
                b2 = (b + AHEAD) % NBUF
                wait_scatter(b2)
                fire_gather(nxt, (b + AHEAD) % G, b2)
        return _
    lax.fori_loop(0, NGRP - 1, _group, None)

    for b in range(NBUF):
        wait_scatter(b)

    plsc.subcore_barrier()

    for k in range(NRCH):
        base = row0 + k * RCH
        pltpu.sync_copy(acc_s.at[pl.ds(base, RCH)], div_v)

        def _div_row(r, _):
            deg = div_v[r, pl.ds(DH, L)]
            recip = 1.0 / deg
            for q in range(DH // L):
                out_v[r, pl.ds(q * L, L)] = div_v[r, pl.ds(q * L, L)] * recip
            return _
        lax.fori_loop(0, RCH, _div_row, None)
        pltpu.sync_copy(out_v, out_hbm.at[pl.ds(base, RCH), pl.ds(c * DH, DH)])


def kernel(features, edge_index):
    dst_r = edge_index[1].reshape(NS, NGRP, G, CHUNK)
    src_r = edge_index[0].reshape(NS, NGRP, G, CHUNK)

    mesh = plsc.VectorSubcoreMesh(core_axis_name="c", subcore_axis_name="s")
    k = pl.kernel(
        _body,
        out_type=(jax.ShapeDtypeStruct((N, D), jnp.float32),
                  jax.ShapeDtypeStruct((NC * N, W), jnp.float32)),
        mesh=mesh,
        scratch_types=[
            pltpu.VMEM((NSETS, G, CHUNK), jnp.int32),
            pltpu.VMEM((NSETS, G, CHUNK), jnp.int32),
            pltpu.VMEM((NBUF, CHUNK, W), jnp.float32),
            pltpu.VMEM((RCH, W), jnp.float32),
            pltpu.VMEM((RCH, DH), jnp.float32),
            pltpu.VMEM((RCH, D), jnp.float32),
            pltpu.VMEM_SHARED((N, W), jnp.float32),
            pltpu.SemaphoreType.DMA((NBUF,)),
            pltpu.SemaphoreType.DMA((NBUF,)),
            pltpu.SemaphoreType.DMA((NSETS,)),
        ],
        compiler_params=pltpu.CompilerParams(use_tc_tiling_on_sc=False),
    )
    out, _table = k(features, dst_r, src_r)
    return out


# speedup vs baseline: 2.4830x; 1.0708x over previous
import jax
import jax.numpy as jnp
from jax import lax
from jax.experimental import pallas as pl
from jax.experimental.pallas import tpu as pltpu
from jax.experimental.pallas import tpu_sc as plsc

N = 10000
D = 128
E = 320000

NC = 2
NS = 16
L = 16

DH = D // NC
W = DH + L
EPT = E // NS
CHUNK = 125
NCHUNK = EPT // CHUNK
RPT = N // NS
RCH = 125
NRCH = RPT // RCH

NBUF = 4
AHEAD = 2
G = 4
NGRP = NCHUNK // G
NSETS = 4


def _body(feat_hbm, dst_hbm, src_hbm, out_hbm, table_hbm,
          dst_idx_v, src_idx_v, rows_v, div_v, out_v, acc_s,
          gsem, ssem, isem):
    c = lax.axis_index("c")
    s = lax.axis_index("s")
    row0 = s * RPT

    def iset(chunk):
        return (chunk // G) % NSETS

    def fire_idx(grp):
        st = grp % NSETS
        pltpu.async_copy(dst_hbm.at[s, grp], dst_idx_v.at[st], isem.at[st])
        pltpu.async_copy(src_hbm.at[s, grp], src_idx_v.at[st], isem.at[st])

    def wait_idx(grp):
        st = grp % NSETS
        pltpu.make_async_copy(dst_hbm.at[s, 0], dst_idx_v.at[st],
                              isem.at[st]).wait()
        pltpu.make_async_copy(src_hbm.at[s, 0], src_idx_v.at[st],
                              isem.at[st]).wait()

    tab_core = table_hbm.at[pl.ds(c * N, N)]

    def fire_gather(ch, row, b):
        pltpu.async_copy(tab_core.at[dst_idx_v.at[iset(ch), row]],
                         rows_v.at[b], gsem.at[b])

    def wait_gather(b):
        pltpu.make_async_copy(tab_core.at[dst_idx_v.at[0, 0]], rows_v.at[b],
                              gsem.at[b]).wait()

    def fire_scatter(ch, row, b):
        pltpu.async_copy(rows_v.at[b], acc_s.at[src_idx_v.at[iset(ch), row]],
                         ssem.at[b], add=True)

    def wait_scatter(b):
        pltpu.make_async_copy(rows_v.at[b], acc_s.at[src_idx_v.at[0, 0]],
                              ssem.at[b]).wait()

    def _ones_row(r, _):
        for b in range(2):
            rows_v[b, r, pl.ds(DH, L)] = jnp.ones((L,), jnp.float32)
        return _
    lax.fori_loop(0, RCH, _ones_row, None)

    def _read_feat(k, b):
        pltpu.async_copy(
            feat_hbm.at[pl.ds(row0 + k * RCH, RCH), pl.ds(c * DH, DH)],
            rows_v.at[b].at[:, pl.ds(0, DH)], gsem.at[b])

    def _wait_feat(b):
        pltpu.make_async_copy(
            feat_hbm.at[pl.ds(0, RCH), pl.ds(0, DH)],
            rows_v.at[b].at[:, pl.ds(0, DH)], gsem.at[b]).wait()

    def _write_tab(k, b):
        pltpu.async_copy(
            rows_v.at[b],
            table_hbm.at[pl.ds(c * N + row0 + k * RCH, RCH)], ssem.at[b])

    def _wait_tab(b):
        pltpu.make_async_copy(
            rows_v.at[b], table_hbm.at[pl.ds(0, RCH)], ssem.at[b]).wait()

    _read_feat(0, 0)
    _read_feat(1, 1)
    for k in range(NRCH):
        b = k % 2
        _wait_feat(b)
        _write_tab(k, b)
        if k + 2 < NRCH:
            _wait_tab(b)
            _read_feat(k + 2, b)
    _wait_tab((NRCH - 2) % 2)
    _wait_tab((NRCH - 1) % 2)

    def _zero_row(r, _):
        for k in range(W // L):
            div_v[r, pl.ds(k * L, L)] = jnp.zeros((L,), jnp.float32)
        return _
    lax.fori_loop(0, RCH, _zero_row, None)
    for k in range(NRCH):
        pltpu.sync_copy(div_v, acc_s.at[pl.ds(row0 + k * RCH, RCH)])

    for grp in range(min(3, NGRP)):
        fire_idx(grp)
    wait_idx(0)

    plsc.subcore_barrier()

    for ch in range(AHEAD):
        fire_gather(ch, ch % G, ch % NBUF)
    for ch in range(NBUF):
        if ch == AHEAD:
            wait_idx(1)
        wait_gather(ch)
        fire_scatter(ch, ch % G, ch)
        if ch + AHEAD >= NBUF:
            wait_scatter((ch + AHEAD) % NBUF)
        fire_gather(ch + AHEAD, (ch + AHEAD) % G, (ch + AHEAD) % NBUF)

    def _group(i, _):
        g = i + 1
        j = g * G

        @pl.when(g + 2 < NGRP)
        def _():
            fire_idx(g + 2)

        for b in range(NBUF):
            ch = j + b
            if b == AHEAD:
                @pl.when(g + 1 < NGRP)
                def _():
                    wait_idx(g + 1)
            wait_gather(b)
            fire_scatter(ch, b, b)
            nxt = ch + AHEAD

            @pl.when(nxt < NCHUNK)
            def _():
                b2 = (b + AHEAD) % NBUF
                wait_scatter(b2)
                fire_gather(nxt, (b + AHEAD) % G, b2)
        return _
    lax.fori_loop(0, NGRP - 1, _group, None)

    for b in range(NBUF):
        wait_scatter(b)

    plsc.subcore_barrier()

    for k in range(NRCH):
        base = row0 + k * RCH
        pltpu.sync_copy(acc_s.at[pl.ds(base, RCH)], div_v)

        def _div_row(r, _):
            deg = div_v[r, pl.ds(DH, L)]
            recip = 1.0 / deg
            for q in range(DH // L):
                out_v[r, pl.ds(q * L, L)] = div_v[r, pl.ds(q * L, L)] * recip
            return _
        lax.fori_loop(0, RCH, _div_row, None)
        pltpu.sync_copy(out_v, out_hbm.at[pl.ds(base, RCH), pl.ds(c * DH, DH)])


def kernel(features, edge_index):
    dst_r = edge_index[1].reshape(NS, NGRP, G, CHUNK)
    src_r = edge_index[0].reshape(NS, NGRP, G, CHUNK)

    mesh = plsc.VectorSubcoreMesh(core_axis_name="c", subcore_axis_name="s")
    k = pl.kernel(
        _body,
        out_type=(jax.ShapeDtypeStruct((N, D), jnp.float32),
                  jax.ShapeDtypeStruct((NC * N, W), jnp.float32)),
        mesh=mesh,
        scratch_types=[
            pltpu.VMEM((NSETS, G, CHUNK), jnp.int32),
            pltpu.VMEM((NSETS, G, CHUNK), jnp.int32),
            pltpu.VMEM((NBUF, CHUNK, W), jnp.float32),
            pltpu.VMEM((RCH, W), jnp.float32),
            pltpu.VMEM((RCH, DH), jnp.float32),
            pltpu.VMEM_SHARED((N, W), jnp.float32),
            pltpu.SemaphoreType.DMA((NBUF,)),
            pltpu.SemaphoreType.DMA((NBUF,)),
            pltpu.SemaphoreType.DMA((NSETS,)),
        ],
        compiler_params=pltpu.CompilerParams(use_tc_tiling_on_sc=False),
    )
    out, _table = k(features, dst_r, src_r)
    return out


# speedup vs baseline: 2.7163x; 1.0939x over previous
import jax
import jax.numpy as jnp
from jax import lax
from jax.experimental import pallas as pl
from jax.experimental.pallas import tpu as pltpu
from jax.experimental.pallas import tpu_sc as plsc

N = 10000
D = 128
E = 320000

NC = 2
NS = 16
L = 16

DH = D // NC
W = DH + L
EPT = E // NS
CHUNK = 125
NCHUNK = EPT // CHUNK
RPT = N // NS
RCH = 125
NRCH = RPT // RCH

NBUF = 4
AHEAD = 3
G = 4
NGRP = NCHUNK // G
NSETS = 4


def _body(feat_hbm, dst_hbm, src_hbm, out_hbm, table_hbm,
          dst_idx_v, src_idx_v, rows_v, div_v, out_v, acc_s,
          gsem, ssem, isem):
    c = lax.axis_index("c")
    s = lax.axis_index("s")
    row0 = s * RPT

    def iset(chunk):
        return (chunk // G) % NSETS

    def fire_idx(grp):
        st = grp % NSETS
        pltpu.async_copy(dst_hbm.at[s, grp], dst_idx_v.at[st], isem.at[st])
        pltpu.async_copy(src_hbm.at[s, grp], src_idx_v.at[st], isem.at[st])

    def wait_idx(grp):
        st = grp % NSETS
        pltpu.make_async_copy(dst_hbm.at[s, 0], dst_idx_v.at[st],
                              isem.at[st]).wait()
        pltpu.make_async_copy(src_hbm.at[s, 0], src_idx_v.at[st],
                              isem.at[st]).wait()

    tab_core = table_hbm.at[pl.ds(c * N, N)]

    def fire_gather(ch, row, b):
        pltpu.async_copy(tab_core.at[dst_idx_v.at[iset(ch), row]],
                         rows_v.at[b], gsem.at[b])

    def wait_gather(b):
        pltpu.make_async_copy(tab_core.at[dst_idx_v.at[0, 0]], rows_v.at[b],
                              gsem.at[b]).wait()

    def fire_scatter(ch, row, b):
        pltpu.async_copy(rows_v.at[b], acc_s.at[src_idx_v.at[iset(ch), row]],
                         ssem.at[b], add=True)

    def wait_scatter(b):
        pltpu.make_async_copy(rows_v.at[b], acc_s.at[src_idx_v.at[0, 0]],
                              ssem.at[b]).wait()

    def _ones_row(r, _):
        for b in range(2):
            rows_v[b, r, pl.ds(DH, L)] = jnp.ones((L,), jnp.float32)
        return _
    lax.fori_loop(0, RCH, _ones_row, None)

    def _read_feat(k, b):
        pltpu.async_copy(
            feat_hbm.at[pl.ds(row0 + k * RCH, RCH), pl.ds(c * DH, DH)],
            rows_v.at[b].at[:, pl.ds(0, DH)], gsem.at[b])

    def _wait_feat(b):
        pltpu.make_async_copy(
            feat_hbm.at[pl.ds(0, RCH), pl.ds(0, DH)],
            rows_v.at[b].at[:, pl.ds(0, DH)], gsem.at[b]).wait()

    def _write_tab(k, b):
        pltpu.async_copy(
            rows_v.at[b],
            table_hbm.at[pl.ds(c * N + row0 + k * RCH, RCH)], ssem.at[b])

    def _wait_tab(b):
        pltpu.make_async_copy(
            rows_v.at[b], table_hbm.at[pl.ds(0, RCH)], ssem.at[b]).wait()

    _read_feat(0, 0)
    _read_feat(1, 1)
    for k in range(NRCH):
        b = k % 2
        _wait_feat(b)
        _write_tab(k, b)
        if k + 2 < NRCH:
            _wait_tab(b)
            _read_feat(k + 2, b)
    _wait_tab((NRCH - 2) % 2)
    _wait_tab((NRCH - 1) % 2)

    def _zero_row(r, _):
        for k in range(W // L):
            div_v[r, pl.ds(k * L, L)] = jnp.zeros((L,), jnp.float32)
        return _
    lax.fori_loop(0, RCH, _zero_row, None)
    for k in range(NRCH):
        pltpu.sync_copy(div_v, acc_s.at[pl.ds(row0 + k * RCH, RCH)])

    for grp in range(min(3, NGRP)):
        fire_idx(grp)
    wait_idx(0)

    plsc.subcore_barrier()

    for ch in range(AHEAD):
        fire_gather(ch, ch % G, ch % NBUF)
    for ch in range(NBUF):
        if ch == G - AHEAD:
            wait_idx(1)
        wait_gather(ch)
        fire_scatter(ch, ch % G, ch)
        if ch + AHEAD >= NBUF:
            wait_scatter((ch + AHEAD) % NBUF)
        fire_gather(ch + AHEAD, (ch + AHEAD) % G, (ch + AHEAD) % NBUF)

    def _group(i, _):
        g = i + 1
        j = g * G

        @pl.when(g + 2 < NGRP)
        def _():
            fire_idx(g + 2)

        for b in range(NBUF):
            ch = j + b
            if b == (G - AHEAD) % G:
                @pl.when(g + 1 < NGRP)
                def _():
                    wait_idx(g + 1)
            wait_gather(b)
            fire_scatter(ch, b, b)
            nxt = ch + AHEAD

            @pl.when(nxt < NCHUNK)
            def _():
                b2 = (b + AHEAD) % NBUF
                wait_scatter(b2)
                fire_gather(nxt, (b + AHEAD) % G, b2)
        return _
    lax.fori_loop(0, NGRP - 1, _group, None)

    for b in range(NBUF):
        wait_scatter(b)

    plsc.subcore_barrier()

    for k in range(NRCH):
        base = row0 + k * RCH
        pltpu.sync_copy(acc_s.at[pl.ds(base, RCH)], div_v)

        def _div_row(r, _):
            deg = div_v[r, pl.ds(DH, L)]
            recip = 1.0 / deg
            for q in range(DH // L):
                out_v[r, pl.ds(q * L, L)] = div_v[r, pl.ds(q * L, L)] * recip
            return _
        lax.fori_loop(0, RCH, _div_row, None)
        pltpu.sync_copy(out_v, out_hbm.at[pl.ds(base, RCH), pl.ds(c * DH, DH)])


def kernel(features, edge_index):
    dst_r = edge_index[1].reshape(NS, NGRP, G, CHUNK)
    src_r = edge_index[0].reshape(NS, NGRP, G, CHUNK)

    mesh = plsc.VectorSubcoreMesh(core_axis_name="c", subcore_axis_name="s")
    k = pl.kernel(
        _body,
        out_type=(jax.ShapeDtypeStruct((N, D), jnp.float32),
                  jax.ShapeDtypeStruct((NC * N, W), jnp.float32)),
        mesh=mesh,
        scratch_types=[
            pltpu.VMEM((NSETS, G, CHUNK), jnp.int32),
            pltpu.VMEM((NSETS, G, CHUNK), jnp.int32),
            pltpu.VMEM((NBUF, CHUNK, W), jnp.float32),
            pltpu.VMEM((RCH, W), jnp.float32),
            pltpu.VMEM((RCH, DH), jnp.float32),
            pltpu.VMEM_SHARED((N, W), jnp.float32),
            pltpu.SemaphoreType.DMA((NBUF,)),
            pltpu.SemaphoreType.DMA((NBUF,)),
            pltpu.SemaphoreType.DMA((NSETS,)),
        ],
        compiler_params=pltpu.CompilerParams(use_tc_tiling_on_sc=False),
    )
    out, _table = k(features, dst_r, src_r)
    return out


# speedup vs baseline: 2.8894x; 1.0637x over previous
import jax
import jax.numpy as jnp
from jax import lax
from jax.experimental import pallas as pl
from jax.experimental.pallas import tpu as pltpu
from jax.experimental.pallas import tpu_sc as plsc

N = 10000
D = 128
E = 320000

NC = 2
NS = 16
L = 16

DH = D // NC
W = DH + L
EPT = E // NS
CHUNK = 125
NCHUNK = EPT // CHUNK
RPT = N // NS
RCH = 125
NRCH = RPT // RCH

NBUF = 5
AHEAD = 4
G = 5
NGRP = NCHUNK // G
NSETS = 4


def _body(feat_hbm, dst_hbm, src_hbm, out_hbm, table_hbm,
          dst_idx_v, src_idx_v, rows_v, div_v, out_v, acc_s,
          gsem, ssem, isem):
    c = lax.axis_index("c")
    s = lax.axis_index("s")
    row0 = s * RPT

    def iset(chunk):
        return (chunk // G) % NSETS

    def fire_idx(grp):
        st = grp % NSETS
        pltpu.async_copy(dst_hbm.at[s, grp], dst_idx_v.at[st], isem.at[st])
        pltpu.async_copy(src_hbm.at[s, grp], src_idx_v.at[st], isem.at[st])

    def wait_idx(grp):
        st = grp % NSETS
        pltpu.make_async_copy(dst_hbm.at[s, 0], dst_idx_v.at[st],
                              isem.at[st]).wait()
        pltpu.make_async_copy(src_hbm.at[s, 0], src_idx_v.at[st],
                              isem.at[st]).wait()

    tab_core = table_hbm.at[pl.ds(c * N, N)]

    def fire_gather(ch, row, b):
        pltpu.async_copy(tab_core.at[dst_idx_v.at[iset(ch), row]],
                         rows_v.at[b], gsem.at[b])

    def wait_gather(b):
        pltpu.make_async_copy(tab_core.at[dst_idx_v.at[0, 0]], rows_v.at[b],
                              gsem.at[b]).wait()

    def fire_scatter(ch, row, b):
        pltpu.async_copy(rows_v.at[b], acc_s.at[src_idx_v.at[iset(ch), row]],
                         ssem.at[b], add=True)

    def wait_scatter(b):
        pltpu.make_async_copy(rows_v.at[b], acc_s.at[src_idx_v.at[0, 0]],
                              ssem.at[b]).wait()

    def _ones_row(r, _):
        for b in range(2):
            rows_v[b, r, pl.ds(DH, L)] = jnp.ones((L,), jnp.float32)
        return _
    lax.fori_loop(0, RCH, _ones_row, None)

    def _read_feat(k, b):
        pltpu.async_copy(
            feat_hbm.at[pl.ds(row0 + k * RCH, RCH), pl.ds(c * DH, DH)],
            rows_v.at[b].at[:, pl.ds(0, DH)], gsem.at[b])

    def _wait_feat(b):
        pltpu.make_async_copy(
            feat_hbm.at[pl.ds(0, RCH), pl.ds(0, DH)],
            rows_v.at[b].at[:, pl.ds(0, DH)], gsem.at[b]).wait()

    def _write_tab(k, b):
        pltpu.async_copy(
            rows_v.at[b],
            table_hbm.at[pl.ds(c * N + row0 + k * RCH, RCH)], ssem.at[b])

    def _wait_tab(b):
        pltpu.make_async_copy(
            rows_v.at[b], table_hbm.at[pl.ds(0, RCH)], ssem.at[b]).wait()

    _read_feat(0, 0)
    _read_feat(1, 1)
    for k in range(NRCH):
        b = k % 2
        _wait_feat(b)
        _write_tab(k, b)
        if k + 2 < NRCH:
            _wait_tab(b)
            _read_feat(k + 2, b)
    _wait_tab((NRCH - 2) % 2)
    _wait_tab((NRCH - 1) % 2)

    def _zero_row(r, _):
        for k in range(W // L):
            div_v[r, pl.ds(k * L, L)] = jnp.zeros((L,), jnp.float32)
        return _
    lax.fori_loop(0, RCH, _zero_row, None)
    for k in range(NRCH):
        pltpu.sync_copy(div_v, acc_s.at[pl.ds(row0 + k * RCH, RCH)])

    for grp in range(min(3, NGRP)):
        fire_idx(grp)
    wait_idx(0)

    plsc.subcore_barrier()

    for ch in range(AHEAD):
        fire_gather(ch, ch % G, ch % NBUF)
    for ch in range(NBUF):
        if ch == G - AHEAD:
            wait_idx(1)
        wait_gather(ch)
        fire_scatter(ch, ch % G, ch)
        if ch + AHEAD >= NBUF:
            wait_scatter((ch + AHEAD) % NBUF)
        fire_gather(ch + AHEAD, (ch + AHEAD) % G, (ch + AHEAD) % NBUF)

    def _group(i, _):
        g = i + 1
        j = g * G

        @pl.when(g + 2 < NGRP)
        def _():
            fire_idx(g + 2)

        for b in range(NBUF):
            ch = j + b
            if b == (G - AHEAD) % G:
                @pl.when(g + 1 < NGRP)
                def _():
                    wait_idx(g + 1)
            wait_gather(b)
            fire_scatter(ch, b, b)
            nxt = ch + AHEAD

            @pl.when(nxt < NCHUNK)
            def _():
                b2 = (b + AHEAD) % NBUF
                wait_scatter(b2)
                fire_gather(nxt, (b + AHEAD) % G, b2)
        return _
    lax.fori_loop(0, NGRP - 1, _group, None)

    for b in range(NBUF):
        wait_scatter(b)

    plsc.subcore_barrier()

    for k in range(NRCH):
        base = row0 + k * RCH
        pltpu.sync_copy(acc_s.at[pl.ds(base, RCH)], div_v)

        def _div_row(r, _):
            deg = div_v[r, pl.ds(DH, L)]
            recip = 1.0 / deg
            for q in range(DH // L):
                out_v[r, pl.ds(q * L, L)] = div_v[r, pl.ds(q * L, L)] * recip
            return _
        lax.fori_loop(0, RCH, _div_row, None)
        pltpu.sync_copy(out_v, out_hbm.at[pl.ds(base, RCH), pl.ds(c * DH, DH)])


def kernel(features, edge_index):
    dst_r = edge_index[1].reshape(NS, NGRP, G, CHUNK)
    src_r = edge_index[0].reshape(NS, NGRP, G, CHUNK)

    mesh = plsc.VectorSubcoreMesh(core_axis_name="c", subcore_axis_name="s")
    k = pl.kernel(
        _body,
        out_type=(jax.ShapeDtypeStruct((N, D), jnp.float32),
                  jax.ShapeDtypeStruct((NC * N, W), jnp.float32)),
        mesh=mesh,
        scratch_types=[
            pltpu.VMEM((NSETS, G, CHUNK), jnp.int32),
            pltpu.VMEM((NSETS, G, CHUNK), jnp.int32),
            pltpu.VMEM((NBUF, CHUNK, W), jnp.float32),
            pltpu.VMEM((RCH, W), jnp.float32),
            pltpu.VMEM((RCH, DH), jnp.float32),
            pltpu.VMEM_SHARED((N, W), jnp.float32),
            pltpu.SemaphoreType.DMA((NBUF,)),
            pltpu.SemaphoreType.DMA((NBUF,)),
            pltpu.SemaphoreType.DMA((NSETS,)),
        ],
        compiler_params=pltpu.CompilerParams(use_tc_tiling_on_sc=False),
    )
    out, _table = k(features, dst_r, src_r)
    return out


# speedup vs baseline: 2.9624x; 1.0253x over previous
import jax
import jax.numpy as jnp
from jax import lax
from jax.experimental import pallas as pl
from jax.experimental.pallas import tpu as pltpu
from jax.experimental.pallas import tpu_sc as plsc

N = 10000
D = 128
E = 320000

NC = 2
NS = 16
L = 16

DH = D // NC
W = DH + L
EPT = E // NS
CHUNK = 100
NCHUNK = EPT // CHUNK
RPT = N // NS
RCH = 125
NRCH = RPT // RCH

NBUF = 8
AHEAD = 6
G = 4
NGRP = NCHUNK // G
NSETS = 6

SZA = [100] * 6 + [25]
OFA = [sum(SZA[:i]) for i in range(7)]


def _body(feat_hbm, dst_hbm, src_hbm, out_hbm, table_hbm,
          dst_idx_v, src_idx_v, rows_v, div_v, acc_s,
          gsem, ssem, isem):
    c = lax.axis_index("c")
    s = lax.axis_index("s")
    row0 = s * RPT

    def iset(chunk):
        return (chunk // G) % NSETS

    def fire_idx(grp):
        st = grp % NSETS
        pltpu.async_copy(dst_hbm.at[s, grp], dst_idx_v.at[st], isem.at[st])
        pltpu.async_copy(src_hbm.at[s, grp], src_idx_v.at[st], isem.at[st])

    def wait_idx(grp):
        st = grp % NSETS
        pltpu.make_async_copy(dst_hbm.at[s, 0], dst_idx_v.at[st],
                              isem.at[st]).wait()
        pltpu.make_async_copy(src_hbm.at[s, 0], src_idx_v.at[st],
                              isem.at[st]).wait()

    tab_core = table_hbm.at[pl.ds(c * N, N)]

    def fire_gather(ch, row, b):
        pltpu.async_copy(tab_core.at[dst_idx_v.at[iset(ch), row]],
                         rows_v.at[b], gsem.at[b])

    def wait_gather(b):
        pltpu.make_async_copy(tab_core.at[dst_idx_v.at[0, 0]], rows_v.at[b],
                              gsem.at[b]).wait()

    def fire_scatter(ch, row, b):
        pltpu.async_copy(rows_v.at[b], acc_s.at[src_idx_v.at[iset(ch), row]],
                         ssem.at[b], add=True)

    def wait_scatter(b):
        pltpu.make_async_copy(rows_v.at[b], acc_s.at[src_idx_v.at[0, 0]],
                              ssem.at[b]).wait()

    def _ones_row(r, _):
        for b in range(2):
            rows_v[b, r, pl.ds(DH, L)] = jnp.ones((L,), jnp.float32)
        return _
    lax.fori_loop(0, CHUNK, _ones_row, None)

    def _read_feat(k, b):
        pltpu.async_copy(
            feat_hbm.at[pl.ds(row0 + OFA[k], SZA[k]), pl.ds(c * DH, DH)],
            rows_v.at[b].at[pl.ds(0, SZA[k]), pl.ds(0, DH)], gsem.at[b])

    def _wait_feat(k, b):
        pltpu.make_async_copy(
            feat_hbm.at[pl.ds(0, SZA[k]), pl.ds(0, DH)],
            rows_v.at[b].at[pl.ds(0, SZA[k]), pl.ds(0, DH)], gsem.at[b]).wait()

    def _write_tab(k, b):
        pltpu.async_copy(
            rows_v.at[b].at[pl.ds(0, SZA[k])],
            table_hbm.at[pl.ds(c * N + row0 + OFA[k], SZA[k])], ssem.at[b])

    def _wait_tab(k, b):
        pltpu.make_async_copy(
            rows_v.at[b].at[pl.ds(0, SZA[k])],
            table_hbm.at[pl.ds(0, SZA[k])], ssem.at[b]).wait()

    _read_feat(0, 0)
    _read_feat(1, 1)
    for k in range(len(SZA)):
        b = k % 2
        _wait_feat(k, b)
        _write_tab(k, b)
        if k + 2 < len(SZA):
            _wait_tab(k, b)
            _read_feat(k + 2, b)
    _wait_tab(len(SZA) - 2, (len(SZA) - 2) % 2)
    _wait_tab(len(SZA) - 1, (len(SZA) - 1) % 2)

    def _zero_row(r, _):
        for k in range(W // L):
            div_v[r, pl.ds(k * L, L)] = jnp.zeros((L,), jnp.float32)
        return _
    lax.fori_loop(0, RCH, _zero_row, None)
    for k in range(NRCH):
        pltpu.sync_copy(div_v, acc_s.at[pl.ds(row0 + k * RCH, RCH)])

    for grp in range(NSETS):
        fire_idx(grp)
    wait_idx(0)
    wait_idx(1)

    plsc.subcore_barrier()

    for ch in range(AHEAD):
        fire_gather(ch, ch % G, ch % NBUF)
    for ch in range(NBUF):
        if ch == 2:
            wait_idx(2)
        if ch == 6:
            wait_idx(3)
        wait_gather(ch)
        fire_scatter(ch, ch % G, ch)
        if ch + AHEAD >= NBUF:
            wait_scatter((ch + AHEAD) % NBUF)
        fire_gather(ch + AHEAD, (ch + AHEAD) % G, (ch + AHEAD) % NBUF)

    def _group(i, _):
        j = NBUF + i * NBUF
        gA = j // G

        @pl.when(gA + 4 < NGRP)
        def _():
            fire_idx(gA + 4)

        for b in range(NBUF):
            ch = j + b
            if b == 2:
                @pl.when(gA + 2 < NGRP)
                def _():
                    wait_idx(gA + 2)
            if b == 6:
                @pl.when(gA + 3 < NGRP)
                def _():
                    wait_idx(gA + 3)
            wait_gather(b)
            fire_scatter(ch, b % G, b)
            nxt = ch + AHEAD

            @pl.when(nxt < NCHUNK)
            def _():
                b2 = (b + AHEAD) % NBUF
                wait_scatter(b2)
                fire_gather(nxt, (b + AHEAD) % G, b2)
            if b == 1:
                @pl.when(gA + 5 < NGRP)
                def _():
                    fire_idx(gA + 5)
        return _
    lax.fori_loop(0, (NCHUNK - NBUF) // NBUF, _group, None)

    for b in range(NBUF):
        wait_scatter(b)

    plsc.subcore_barrier()

    for k in range(NRCH):
        base = row0 + k * RCH
        pltpu.sync_copy(acc_s.at[pl.ds(base, RCH)], div_v)

        def _div_row(r, _):
            deg = div_v[r, pl.ds(DH, L)]
            recip = 1.0 / deg
            for q in range(DH // L):
                div_v[r, pl.ds(q * L, L)] = div_v[r, pl.ds(q * L, L)] * recip
            return _
        lax.fori_loop(0, RCH, _div_row, None)
        pltpu.sync_copy(div_v.at[:, pl.ds(0, DH)],
                        out_hbm.at[pl.ds(base, RCH), pl.ds(c * DH, DH)])


def kernel(features, edge_index):
    dst_r = edge_index[1].reshape(NS, NGRP, G, CHUNK)
    src_r = edge_index[0].reshape(NS, NGRP, G, CHUNK)

    mesh = plsc.VectorSubcoreMesh(core_axis_name="c", subcore_axis_name="s")
    k = pl.kernel(
        _body,
        out_type=(jax.ShapeDtypeStruct((N, D), jnp.float32),
                  jax.ShapeDtypeStruct((NC * N, W), jnp.float32)),
        mesh=mesh,
        scratch_types=[
            pltpu.VMEM((NSETS, G, CHUNK), jnp.int32),
            pltpu.VMEM((NSETS, G, CHUNK), jnp.int32),
            pltpu.VMEM((NBUF, CHUNK, W), jnp.float32),
            pltpu.VMEM((RCH, W), jnp.float32),
            pltpu.VMEM_SHARED((N, W), jnp.float32),
            pltpu.SemaphoreType.DMA((NBUF,)),
            pltpu.SemaphoreType.DMA((NBUF,)),
            pltpu.SemaphoreType.DMA((NSETS,)),
        ],
        compiler_params=pltpu.CompilerParams(use_tc_tiling_on_sc=False),
    )
    out, _table = k(features, dst_r, src_r)
    return out
